# baseline (speedup 1.0000x reference)
import functools

import jax
import jax.numpy as jnp
import numpy as np
from jax.experimental import pallas as pl
from jax.experimental.pallas import tpu as pltpu

DT = 0.01
DH = 0.1
R2 = 1.5 * 1.5
COORDS_WEIGHT = 1.0
NP = 32


def _silu(x):
    return 0.5 * x * (1.0 + jnp.tanh(0.5 * x))


def _enflow_kernel(n_layers, n_atoms, bb,
                   h_ref, pos_ref, vel_ref, g_ref, bmask_ref,
                   wa_ref, wb_ref, wr_ref, be1_ref,
                   we2_ref, be2_ref, wc1_ref, bc1_ref, wc2_ref,
                   wn1h_ref, wn1a_ref, bn1_ref, wn2_ref, bn2_ref,
                   ws_ref, bs_ref,
                   h_out, pos_out, vel_out, g_out, s_out):
    nf = h_ref.shape[-1]
    ni = n_atoms
    npad = NP - n_atoms

    def padj(x):
        z = jnp.zeros(x.shape[:1] + (npad,) + x.shape[2:], x.dtype)
        return jnp.concatenate([x, z], axis=1)

    h = padj(h_ref[...])
    pos = padj(pos_ref[...])
    vel = padj(vel_ref[...])
    g = padj(g_ref[...])
    bmask = bmask_ref[...]

    dot = functools.partial(jnp.dot, preferred_element_type=jnp.float32)
    hi = jax.lax.Precision.HIGHEST

    s_acc = jnp.zeros((bb * NP, 1), jnp.float32)

    for l in range(n_layers):
        diff = pos[:, :ni, None, :] - pos[:, None, :, :]
        radial = jnp.sum(diff * diff, axis=-1, keepdims=True)
        maskf = jnp.where(radial < R2, bmask, 0.0)

        hf2 = h.reshape(bb * NP, nf)
        a = dot(hf2, wa_ref[l]) + be1_ref[l]
        b2 = dot(hf2, wb_ref[l])
        a4 = a.reshape(bb, NP, nf)[:, :ni, None, :]
        b4 = b2.reshape(bb, NP, nf)[:, None, :, :]
        pre = a4 + b4 + radial * wr_ref[l]
        m = _silu(pre.reshape(bb * ni * NP, nf))
        m = _silu(dot(m, we2_ref[l]) + be2_ref[l])
        m4 = m.reshape(bb, ni, NP, nf) * maskf
        agg = jnp.sum(m4, axis=2)
        mflat = m4.reshape(bb * ni * NP, nf)
        c1 = _silu(dot(mflat, wc1_ref[l]) + bc1_ref[l])
        cm = dot(c1, wc2_ref[l])
        cm4 = cm.reshape(bb, ni, NP, 1) * maskf
        force = jnp.sum(diff * cm4, axis=2) * COORDS_WEIGHT

        aggp = jnp.pad(agg, ((0, 0), (0, npad), (0, 0)))
        forcep = jnp.pad(force, ((0, 0), (0, npad), (0, 0)))
        aggf = aggp.reshape(bb * NP, nf)
        hn = _silu(dot(hf2, wn1h_ref[l]) + dot(aggf, wn1a_ref[l]) + bn1_ref[l])
        hforce = dot(hn, wn2_ref[l]) + bn2_ref[l]
        s = dot(aggf, ws_ref[l]) + bs_ref[l]

        s3 = s.reshape(bb, NP, 1)
        vel = jnp.exp(s3) * vel + forcep * DT
        pos = pos + vel * DT
        g = g + hforce.reshape(bb, NP, nf) * DH
        h = h + g * DH
        s_acc = s_acc + s

    h_out[...] = h[:, :ni]
    pos_out[...] = pos[:, :ni]
    vel_out[...] = vel[:, :ni]
    g_out[...] = g[:, :ni]
    s_out[...] = s_acc.reshape(bb, NP, 1)[:, :ni]


def kernel(h, pos, vel, g, params):
    B, N, nf = h.shape
    n_layers = len(params)
    bb = 8

    i_idx = np.arange(N)[:, None]
    j_idx = np.arange(NP)[None, :]
    bmask_np = ((i_idx != j_idx) & (j_idx < N)).astype(np.float32)[:, :, None]
    bmask = jnp.asarray(bmask_np)

    st = lambda name: jnp.stack([p[name] for p in params])
    we1 = st("We1")
    wa = we1[:, :nf]
    wb = we1[:, nf:2 * nf]
    wr = we1[:, 2 * nf:]
    be1 = st("be1")[:, None, :]
    we2 = st("We2")
    be2 = st("be2")[:, None, :]
    wc1 = st("Wc1")
    bc1 = st("bc1")[:, None, :]
    wc2 = st("Wc2")
    wn1 = st("Wn1")
    wn1h = wn1[:, :nf]
    wn1a = wn1[:, nf:]
    bn1 = st("bn1")[:, None, :]
    wn2 = st("Wn2")
    bn2 = st("bn2")[:, None, :]
    ws = st("Ws")
    bs = st("bs")[:, :, None]

    def wspec(x):
        return pl.BlockSpec(x.shape, lambda i: (0,) * x.ndim)

    def bspec(last):
        return pl.BlockSpec((bb, N, last), lambda i: (i, 0, 0))

    weights = (wa, wb, wr, be1, we2, be2, wc1, bc1, wc2,
               wn1h, wn1a, bn1, wn2, bn2, ws, bs)

    outs = pl.pallas_call(
        functools.partial(_enflow_kernel, n_layers, N, bb),
        grid=(B // bb,),
        in_specs=[bspec(nf), bspec(3), bspec(3), bspec(nf), wspec(bmask)]
                 + [wspec(w) for w in weights],
        out_specs=[bspec(nf), bspec(3), bspec(3), bspec(nf), bspec(1)],
        out_shape=[
            jax.ShapeDtypeStruct((B, N, nf), jnp.float32),
            jax.ShapeDtypeStruct((B, N, 3), jnp.float32),
            jax.ShapeDtypeStruct((B, N, 3), jnp.float32),
            jax.ShapeDtypeStruct((B, N, nf), jnp.float32),
            jax.ShapeDtypeStruct((B, N, 1), jnp.float32),
        ],
        compiler_params=pltpu.CompilerParams(
            dimension_semantics=("parallel",)),
    )(h, pos, vel, g, bmask, *weights)

    h_o, pos_o, vel_o, g_o, s_o = outs
    ldj = jnp.sum(s_o)
    return (h_o, pos_o, vel_o, g_o, ldj)


# speedup vs baseline: 1.0016x; 1.0016x over previous
import functools

import jax
import jax.numpy as jnp
import numpy as np
from jax.experimental import pallas as pl
from jax.experimental.pallas import tpu as pltpu

DT = 0.01
DH = 0.1
R2 = 1.5 * 1.5
COORDS_WEIGHT = 1.0
NP = 32


def _silu(x):
    return 0.5 * x * (1.0 + jnp.tanh(0.5 * x))


def _enflow_kernel(n_layers, n_atoms, bb,
                   h_ref, pos_ref, vel_ref, g_ref, bmask_ref,
                   wa_ref, wb_ref, wr_ref, be1_ref,
                   we2_ref, be2_ref, wc1_ref, bc1_ref, wc2_ref,
                   wn1h_ref, wn1a_ref, bn1_ref, wn2_ref, bn2_ref,
                   ws_ref, bs_ref,
                   h_out, pos_out, vel_out, g_out, s_out):
    nf = h_ref.shape[-1]
    ni = n_atoms
    npad = NP - n_atoms

    def padj(x):
        z = jnp.zeros(x.shape[:1] + (npad,) + x.shape[2:], x.dtype)
        return jnp.concatenate([x, z], axis=1)

    h = padj(h_ref[...])
    pos = padj(pos_ref[...])
    vel = padj(vel_ref[...])
    g = padj(g_ref[...])
    bmask = bmask_ref[...]

    dot = functools.partial(jnp.dot, preferred_element_type=jnp.float32)

    s_acc = jnp.zeros((bb * NP, 1), jnp.float32)

    for l in range(n_layers):
        diff = pos[:, :ni, None, :] - pos[:, None, :, :]
        radial = jnp.sum(diff * diff, axis=-1, keepdims=True)
        maskf = jnp.where(radial < R2, bmask, 0.0)

        hf2 = h.reshape(bb * NP, nf)
        a = dot(hf2, wa_ref[l]) + be1_ref[l]
        b2 = dot(hf2, wb_ref[l])
        a4 = a.reshape(bb, NP, nf)[:, :ni, None, :]
        b4 = b2.reshape(bb, NP, nf)[:, None, :, :]
        pre = a4 + b4 + radial * wr_ref[l]
        m = _silu(pre.reshape(bb * ni * NP, nf))
        m = _silu(dot(m, we2_ref[l]) + be2_ref[l])
        m4 = m.reshape(bb, ni, NP, nf) * maskf
        agg = jnp.sum(m4, axis=2)
        mflat = m4.reshape(bb * ni * NP, nf)
        c1 = _silu(dot(mflat, wc1_ref[l]) + bc1_ref[l])
        cm = dot(c1, wc2_ref[l])
        cm4 = cm.reshape(bb, ni, NP, 1) * maskf
        force = jnp.sum(diff * cm4, axis=2) * COORDS_WEIGHT

        aggp = jnp.pad(agg, ((0, 0), (0, npad), (0, 0)))
        forcep = jnp.pad(force, ((0, 0), (0, npad), (0, 0)))
        aggf = aggp.reshape(bb * NP, nf)
        hn = _silu(dot(hf2, wn1h_ref[l]) + dot(aggf, wn1a_ref[l]) + bn1_ref[l])
        hforce = dot(hn, wn2_ref[l]) + bn2_ref[l]
        s = dot(aggf, ws_ref[l]) + bs_ref[l]

        s3 = s.reshape(bb, NP, 1)
        vel = jnp.exp(s3) * vel + forcep * DT
        pos = pos + vel * DT
        g = g + hforce.reshape(bb, NP, nf) * DH
        h = h + g * DH
        s_acc = s_acc + s

    h_out[...] = h[:, :ni]
    pos_out[...] = pos[:, :ni]
    vel_out[...] = vel[:, :ni]
    g_out[...] = g[:, :ni]
    s_out[...] = s_acc.reshape(bb, NP, 1)[:, :ni]


def kernel(h, pos, vel, g, params):
    B, N, nf = h.shape
    n_layers = len(params)
    bb = 16

    i_idx = np.arange(N)[:, None]
    j_idx = np.arange(NP)[None, :]
    bmask_np = ((i_idx != j_idx) & (j_idx < N)).astype(np.float32)[:, :, None]
    bmask = jnp.asarray(bmask_np)

    st = lambda name: jnp.stack([p[name] for p in params])
    we1 = st("We1")
    wa = we1[:, :nf]
    wb = we1[:, nf:2 * nf]
    wr = we1[:, 2 * nf:]
    be1 = st("be1")[:, None, :]
    we2 = st("We2")
    be2 = st("be2")[:, None, :]
    wc1 = st("Wc1")
    bc1 = st("bc1")[:, None, :]
    wc2 = st("Wc2")
    wn1 = st("Wn1")
    wn1h = wn1[:, :nf]
    wn1a = wn1[:, nf:]
    bn1 = st("bn1")[:, None, :]
    wn2 = st("Wn2")
    bn2 = st("bn2")[:, None, :]
    ws = st("Ws")
    bs = st("bs")[:, :, None]

    def wspec(x):
        return pl.BlockSpec(x.shape, lambda i: (0,) * x.ndim)

    def bspec(last):
        return pl.BlockSpec((bb, N, last), lambda i: (i, 0, 0))

    weights = (wa, wb, wr, be1, we2, be2, wc1, bc1, wc2,
               wn1h, wn1a, bn1, wn2, bn2, ws, bs)

    outs = pl.pallas_call(
        functools.partial(_enflow_kernel, n_layers, N, bb),
        grid=(B // bb,),
        in_specs=[bspec(nf), bspec(3), bspec(3), bspec(nf), wspec(bmask)]
                 + [wspec(w) for w in weights],
        out_specs=[bspec(nf), bspec(3), bspec(3), bspec(nf), bspec(1)],
        out_shape=[
            jax.ShapeDtypeStruct((B, N, nf), jnp.float32),
            jax.ShapeDtypeStruct((B, N, 3), jnp.float32),
            jax.ShapeDtypeStruct((B, N, 3), jnp.float32),
            jax.ShapeDtypeStruct((B, N, nf), jnp.float32),
            jax.ShapeDtypeStruct((B, N, 1), jnp.float32),
        ],
        compiler_params=pltpu.CompilerParams(
            dimension_semantics=("parallel",)),
    )(h, pos, vel, g, bmask, *weights)

    h_o, pos_o, vel_o, g_o, s_o = outs
    ldj = jnp.sum(s_o)
    return (h_o, pos_o, vel_o, g_o, ldj)
